# Initial kernel scaffold; baseline (speedup 1.0000x reference)
import jax
import jax.numpy as jnp
from jax.experimental import pallas as pl


def kernel(mlvl_feats, points, ori_points, img, lidar2camera, lidar2image, cam_intrinsic, cam_2_lidar, img_aug_matrix, lidar_aug_matrix, img_metas, conv_w, conv_b, bn_gamma, bn_beta, bn_mean, bn_var):
    raise NotImplementedError("write your pallas kernel here")


# speedup vs baseline: 4.4288x; 4.4288x over previous
import functools

import jax
import jax.numpy as jnp
from jax import lax
from jax.experimental import pallas as pl
from jax.experimental.pallas import tpu as pltpu
from jax.experimental.pallas import tpu_sc as plsc

NCAM = 6
CIN = 256
COUT = 80
HF, WF = 32, 88
HW = HF * WF
NX = NY = 200
NZ = 4
NPIX = NX * NY
NWORK = 32
PIX_PER_W = 1280
NPIX_PAD = NWORK * PIX_PER_W
CHUNK = 128
NCHUNK = PIX_PER_W // CHUNK
SENTINEL = NCAM * HW
TROWS = 67 * 256
CPAD = 128
BLK_P = 4096


def _pf_body(feats_ref, cw_ref, out_ref):
    b = pl.program_id(0)
    a = feats_ref[0]
    res = lax.dot_general(a, cw_ref[...], (((0,), (0,)), ((), ())),
                          preferred_element_type=jnp.float32)
    out_ref[...] = jnp.where(b < 66, res, 0.0)


def _pf_table(feats_r, cw_s):
    return pl.pallas_call(
        _pf_body,
        grid=(67,),
        in_specs=[
            pl.BlockSpec((1, CIN, 256),
                         lambda b: (jnp.minimum(b // 11, 5), 0,
                                    jnp.minimum(b % 11, 10))),
            pl.BlockSpec((CIN, CPAD), lambda b: (0, 0)),
        ],
        out_specs=pl.BlockSpec((256, CPAD), lambda b: (b, 0)),
        out_shape=jax.ShapeDtypeStruct((TROWS, CPAD), jnp.float32),
    )(feats_r, cw_s)


def _idx_body(u_ref, v_ref, z_ref, out_ref):
    idx = jnp.full((NZ, BLK_P), SENTINEL, jnp.int32)
    for cam in range(NCAM):
        u = u_ref[cam]
        v = v_ref[cam]
        zc = z_ref[cam]
        ur = jnp.round(u * 0.125)
        vr = jnp.round(v * 0.125)
        valid = (ur >= 0.0) & (vr >= 0.0) & (ur < WF) & (vr < HF) & (zc > 0.0)
        urc = jnp.clip(ur, 0.0, WF - 1.0)
        vrc = jnp.clip(vr, 0.0, HF - 1.0)
        cand = cam * HW + vrc.astype(jnp.int32) * WF + urc.astype(jnp.int32)
        idx = jnp.where(valid, cand, idx)
    out_ref[...] = idx


def _idx_map(u, v, z):
    spec = pl.BlockSpec((NCAM, NZ, BLK_P), lambda t: (0, 0, t))
    return pl.pallas_call(
        _idx_body,
        grid=(NPIX_PAD // BLK_P,),
        in_specs=[spec, spec, spec],
        out_specs=pl.BlockSpec((NZ, BLK_P), lambda t: (0, t)),
        out_shape=jax.ShapeDtypeStruct((NZ, NPIX_PAD), jnp.int32),
    )(u, v, z)


_SC_MESH = plsc.VectorSubcoreMesh(core_axis_name="c", subcore_axis_name="s",
                                  num_cores=2, num_subcores=16)


@functools.partial(
    pl.kernel,
    out_type=jax.ShapeDtypeStruct((NZ, NPIX_PAD, CPAD), jnp.float32),
    mesh=_SC_MESH,
    scratch_types=[
        pltpu.VMEM((NZ, CHUNK), jnp.int32),
        pltpu.VMEM((NZ, CHUNK, CPAD), jnp.float32),
        pltpu.SemaphoreType.DMA,
    ],
)
def _sc_gather(table_hbm, idx_hbm, g_hbm, idx_v, rows_v, sem):
    wid = lax.axis_index("s") * 2 + lax.axis_index("c")
    for t in range(NCHUNK):
        base = wid * PIX_PER_W + t * CHUNK
        pltpu.sync_copy(idx_hbm.at[:, pl.ds(base, CHUNK)], idx_v)
        cps = [pltpu.async_copy(table_hbm.at[idx_v.at[z]], rows_v.at[z], sem)
               for z in range(NZ)]
        for z in range(NZ):
            cps[z].wait()
        for z in range(NZ):
            pltpu.sync_copy(rows_v.at[z], g_hbm.at[z, pl.ds(base, CHUNK)])


def _combine_body(g_ref, w_ref, eye_ref, bias_ref, out_ref):
    g = g_ref[...]
    w = w_ref[...]
    acc = jnp.sum(g * w[:, :, None], axis=0)
    out_t = lax.dot_general(eye_ref[...], acc, (((1,), (1,)), ((), ())),
                            preferred_element_type=jnp.float32)
    out_ref[...] = jnp.maximum(out_t + bias_ref[...], 0.0)


def _combine(g, w_pad, eye, bias_col):
    return pl.pallas_call(
        _combine_body,
        grid=(NPIX_PAD // BLK_P,),
        in_specs=[
            pl.BlockSpec((NZ, BLK_P, CPAD), lambda t: (0, t, 0)),
            pl.BlockSpec((NZ, BLK_P), lambda t: (0, t)),
            pl.BlockSpec((COUT, CPAD), lambda t: (0, 0)),
            pl.BlockSpec((COUT, 1), lambda t: (0, 0)),
        ],
        out_specs=pl.BlockSpec((COUT, BLK_P), lambda t: (0, t)),
        out_shape=jax.ShapeDtypeStruct((COUT, NPIX), jnp.float32),
    )(g, w_pad, eye, bias_col)


def kernel(mlvl_feats, points, ori_points, img, lidar2camera, lidar2image,
           cam_intrinsic, cam_2_lidar, img_aug_matrix, lidar_aug_matrix,
           img_metas, conv_w, conv_b, bn_gamma, bn_beta, bn_mean, bn_var):
    iam = img_aug_matrix[0]
    iam_r = iam.at[..., -1].set(0.0)
    proj = jnp.matmul(jnp.matmul(iam_r, lidar2image[0]),
                      lidar_aug_matrix[0])[:, :3, :]
    iam_t = iam[..., -1]

    pixv = jnp.arange(NPIX_PAD, dtype=jnp.int32)
    gx = (pixv // NY).astype(jnp.float32) * 0.5 - 50.0
    gy = (pixv % NY).astype(jnp.float32) * 0.5 - 50.0
    gz = jnp.arange(NZ, dtype=jnp.float32) * 1.5 - 4.0
    xf = jnp.broadcast_to(gx[None, :], (NZ, NPIX_PAD)).reshape(-1)
    yf = jnp.broadcast_to(gy[None, :], (NZ, NPIX_PAD)).reshape(-1)
    zf = jnp.broadcast_to(gz[:, None], (NZ, NPIX_PAD)).reshape(-1)
    pts = jnp.stack([xf, yf, zf, jnp.ones_like(xf)], axis=0)
    p2i = jnp.einsum('nij,jk->nik', proj, pts)
    zc = p2i[:, 2]
    uc = p2i[:, 0] / zc + iam_t[:, 0:1]
    vc = p2i[:, 1] / zc + iam_t[:, 1:2]
    u3 = uc.reshape(NCAM, NZ, NPIX_PAD)
    v3 = vc.reshape(NCAM, NZ, NPIX_PAD)
    z3 = zc.reshape(NCAM, NZ, NPIX_PAD)

    s = bn_gamma / jnp.sqrt(bn_var + 1e-5)
    bias_eff = (conv_b - bn_mean) * s + bn_beta
    cw_s = conv_w.T * s[None, :]
    cw_s = jnp.pad(cw_s, ((0, 0), (0, CPAD - COUT)))

    feats_r = mlvl_feats[0].reshape(NCAM, CIN, HW)
    w_pad = jnp.pad(points[0].reshape(NZ, NPIX),
                    ((0, 0), (0, NPIX_PAD - NPIX)))

    table = _pf_table(feats_r, cw_s)
    idx = _idx_map(u3, v3, z3)
    g = _sc_gather(table, idx)
    eye = jnp.eye(COUT, CPAD, dtype=jnp.float32)
    y = _combine(g, w_pad, eye, bias_eff[:, None])
    return y.reshape(1, COUT, NX, NY)


# speedup vs baseline: 4.5394x; 1.0250x over previous
import functools

import jax
import jax.numpy as jnp
from jax import lax
from jax.experimental import pallas as pl
from jax.experimental.pallas import tpu as pltpu
from jax.experimental.pallas import tpu_sc as plsc

NCAM = 6
CIN = 256
COUT = 80
HF, WF = 32, 88
HW = HF * WF
NX = NY = 200
NZ = 4
NPIX = NX * NY
NWORK = 32
PIX_PER_W = 1280
NPIX_PAD = NWORK * PIX_PER_W
CHUNK = 128
NCHUNK = PIX_PER_W // CHUNK
SENTINEL = NCAM * HW
TROWS = 67 * 256
CPAD = 128
BLK_P = 4096


def _pf_body(feats_ref, cw_ref, out_ref):
    b = pl.program_id(0)
    a = feats_ref[0]
    res = lax.dot_general(a, cw_ref[...], (((0,), (0,)), ((), ())),
                          preferred_element_type=jnp.float32)
    out_ref[...] = jnp.where(b < 66, res, 0.0)


def _pf_table(feats_r, cw_s):
    return pl.pallas_call(
        _pf_body,
        grid=(67,),
        in_specs=[
            pl.BlockSpec((1, CIN, 256),
                         lambda b: (jnp.minimum(b // 11, 5), 0,
                                    jnp.minimum(b % 11, 10))),
            pl.BlockSpec((CIN, CPAD), lambda b: (0, 0)),
        ],
        out_specs=pl.BlockSpec((256, CPAD), lambda b: (b, 0)),
        out_shape=jax.ShapeDtypeStruct((TROWS, CPAD), jnp.float32),
    )(feats_r, cw_s)


def _idx_body(u_ref, v_ref, z_ref, out_ref):
    idx = jnp.full((NZ, BLK_P), SENTINEL, jnp.int32)
    for cam in range(NCAM):
        u = u_ref[cam]
        v = v_ref[cam]
        zc = z_ref[cam]
        ur = jnp.round(u * 0.125)
        vr = jnp.round(v * 0.125)
        valid = (ur >= 0.0) & (vr >= 0.0) & (ur < WF) & (vr < HF) & (zc > 0.0)
        urc = jnp.clip(ur, 0.0, WF - 1.0)
        vrc = jnp.clip(vr, 0.0, HF - 1.0)
        cand = cam * HW + vrc.astype(jnp.int32) * WF + urc.astype(jnp.int32)
        idx = jnp.where(valid, cand, idx)
    out_ref[...] = idx


def _idx_map(u, v, z):
    spec = pl.BlockSpec((NCAM, NZ, BLK_P), lambda t: (0, 0, t))
    return pl.pallas_call(
        _idx_body,
        grid=(NPIX_PAD // BLK_P,),
        in_specs=[spec, spec, spec],
        out_specs=pl.BlockSpec((NZ, BLK_P), lambda t: (0, t)),
        out_shape=jax.ShapeDtypeStruct((NZ, NPIX_PAD), jnp.int32),
    )(u, v, z)


_SC_MESH = plsc.VectorSubcoreMesh(core_axis_name="c", subcore_axis_name="s",
                                  num_cores=2, num_subcores=16)


SLOTS_PER_W = NZ * NPIX_PAD // NWORK
NT = SLOTS_PER_W // CHUNK
NB = 4


@functools.partial(
    pl.kernel,
    out_type=jax.ShapeDtypeStruct((NZ, NPIX_PAD, CPAD), jnp.float32),
    mesh=_SC_MESH,
    scratch_types=[
        pltpu.VMEM((SLOTS_PER_W,), jnp.int32),
        pltpu.VMEM((NB, CHUNK, CPAD), jnp.float32),
        pltpu.SemaphoreType.DMA,
        pltpu.SemaphoreType.DMA,
    ],
)
def _sc_gather(table_hbm, idx_hbm, g_hbm, idx_v, rows_v, gsem, osem):
    wid = lax.axis_index("s") * 2 + lax.axis_index("c")
    z0 = wid // 8
    pixbase = (wid % 8) * SLOTS_PER_W
    pltpu.sync_copy(idx_hbm.at[z0, pl.ds(pixbase, SLOTS_PER_W)], idx_v)
    gcps = [None] * NB
    ocps = [None] * NB

    def fire(t):
        b = t % NB
        if ocps[b] is not None:
            ocps[b].wait()
            ocps[b] = None
        gcps[b] = pltpu.async_copy(
            table_hbm.at[idx_v.at[pl.ds(t * CHUNK, CHUNK)]], rows_v.at[b], gsem)

    for t in range(NB - 1):
        fire(t)
    for t in range(NT):
        b = t % NB
        if t + NB - 1 < NT:
            fire(t + NB - 1)
        gcps[b].wait()
        ocps[b] = pltpu.async_copy(
            rows_v.at[b], g_hbm.at[z0, pl.ds(pixbase + t * CHUNK, CHUNK)], osem)
    for b in range(NB):
        if ocps[b] is not None:
            ocps[b].wait()


def _combine_body(g_ref, w_ref, eye_ref, bias_ref, out_ref):
    g = g_ref[...]
    w = w_ref[...]
    acc = jnp.sum(g * w[:, :, None], axis=0)
    out_t = lax.dot_general(eye_ref[...], acc, (((1,), (1,)), ((), ())),
                            preferred_element_type=jnp.float32)
    out_ref[...] = jnp.maximum(out_t + bias_ref[...], 0.0)


def _combine(g, w_pad, eye, bias_col):
    return pl.pallas_call(
        _combine_body,
        grid=(NPIX_PAD // BLK_P,),
        in_specs=[
            pl.BlockSpec((NZ, BLK_P, CPAD), lambda t: (0, t, 0)),
            pl.BlockSpec((NZ, BLK_P), lambda t: (0, t)),
            pl.BlockSpec((COUT, CPAD), lambda t: (0, 0)),
            pl.BlockSpec((COUT, 1), lambda t: (0, 0)),
        ],
        out_specs=pl.BlockSpec((COUT, BLK_P), lambda t: (0, t)),
        out_shape=jax.ShapeDtypeStruct((COUT, NPIX), jnp.float32),
    )(g, w_pad, eye, bias_col)


def kernel(mlvl_feats, points, ori_points, img, lidar2camera, lidar2image,
           cam_intrinsic, cam_2_lidar, img_aug_matrix, lidar_aug_matrix,
           img_metas, conv_w, conv_b, bn_gamma, bn_beta, bn_mean, bn_var):
    iam = img_aug_matrix[0]
    iam_r = iam.at[..., -1].set(0.0)
    proj = jnp.matmul(jnp.matmul(iam_r, lidar2image[0]),
                      lidar_aug_matrix[0])[:, :3, :]
    iam_t = iam[..., -1]

    pixv = jnp.arange(NPIX_PAD, dtype=jnp.int32)
    gx = (pixv // NY).astype(jnp.float32) * 0.5 - 50.0
    gy = (pixv % NY).astype(jnp.float32) * 0.5 - 50.0
    gz = jnp.arange(NZ, dtype=jnp.float32) * 1.5 - 4.0
    xf = jnp.broadcast_to(gx[None, :], (NZ, NPIX_PAD)).reshape(-1)
    yf = jnp.broadcast_to(gy[None, :], (NZ, NPIX_PAD)).reshape(-1)
    zf = jnp.broadcast_to(gz[:, None], (NZ, NPIX_PAD)).reshape(-1)
    pts = jnp.stack([xf, yf, zf, jnp.ones_like(xf)], axis=0)
    p2i = jnp.einsum('nij,jk->nik', proj, pts)
    zc = p2i[:, 2]
    uc = p2i[:, 0] / zc + iam_t[:, 0:1]
    vc = p2i[:, 1] / zc + iam_t[:, 1:2]
    u3 = uc.reshape(NCAM, NZ, NPIX_PAD)
    v3 = vc.reshape(NCAM, NZ, NPIX_PAD)
    z3 = zc.reshape(NCAM, NZ, NPIX_PAD)

    s = bn_gamma / jnp.sqrt(bn_var + 1e-5)
    bias_eff = (conv_b - bn_mean) * s + bn_beta
    cw_s = conv_w.T * s[None, :]
    cw_s = jnp.pad(cw_s, ((0, 0), (0, CPAD - COUT)))

    feats_r = mlvl_feats[0].reshape(NCAM, CIN, HW)
    w_pad = jnp.pad(points[0].reshape(NZ, NPIX),
                    ((0, 0), (0, NPIX_PAD - NPIX)))

    table = _pf_table(feats_r, cw_s)
    idx = _idx_map(u3, v3, z3)
    g = _sc_gather(table, idx)
    eye = jnp.eye(COUT, CPAD, dtype=jnp.float32)
    y = _combine(g, w_pad, eye, bias_eff[:, None])
    return y.reshape(1, COUT, NX, NY)


# speedup vs baseline: 10.1136x; 2.2279x over previous
import functools

import jax
import jax.numpy as jnp
from jax import lax
from jax.experimental import pallas as pl
from jax.experimental.pallas import tpu as pltpu
from jax.experimental.pallas import tpu_sc as plsc

NCAM = 6
CIN = 256
COUT = 80
HF, WF = 32, 88
HW = HF * WF
NX = NY = 200
NZ = 4
NPIX = NX * NY
NWORK = 32
PIX_PER_W = 1280
NPIX_PAD = NWORK * PIX_PER_W
CHUNK = 128
NCHUNK = PIX_PER_W // CHUNK
SENTINEL = NCAM * HW
TROWS = 67 * 256
CPAD = 128
BLK_P = 4096


def _pf_body(feats_ref, cw_ref, out_ref):
    b = pl.program_id(0)
    a = feats_ref[0]
    res = lax.dot_general(a, cw_ref[...], (((0,), (0,)), ((), ())),
                          preferred_element_type=jnp.float32)
    out_ref[...] = jnp.where(b < 66, res, 0.0)


def _pf_table(feats_r, cw_s):
    return pl.pallas_call(
        _pf_body,
        grid=(67,),
        in_specs=[
            pl.BlockSpec((1, CIN, 256),
                         lambda b: (jnp.minimum(b // 11, 5), 0,
                                    jnp.minimum(b % 11, 10))),
            pl.BlockSpec((CIN, CPAD), lambda b: (0, 0)),
        ],
        out_specs=pl.BlockSpec((256, CPAD), lambda b: (b, 0)),
        out_shape=jax.ShapeDtypeStruct((TROWS, CPAD), jnp.float32),
    )(feats_r, cw_s)


def _idx_body(u_ref, v_ref, z_ref, out_ref):
    idx = jnp.full((NZ, BLK_P), SENTINEL, jnp.int32)
    for cam in range(NCAM):
        u = u_ref[cam]
        v = v_ref[cam]
        zc = z_ref[cam]
        ur = jnp.round(u * 0.125)
        vr = jnp.round(v * 0.125)
        valid = (ur >= 0.0) & (vr >= 0.0) & (ur < WF) & (vr < HF) & (zc > 0.0)
        urc = jnp.clip(ur, 0.0, WF - 1.0)
        vrc = jnp.clip(vr, 0.0, HF - 1.0)
        cand = cam * HW + vrc.astype(jnp.int32) * WF + urc.astype(jnp.int32)
        idx = jnp.where(valid, cand, idx)
    out_ref[...] = idx


def _idx_map(u, v, z):
    spec = pl.BlockSpec((NCAM, NZ, BLK_P), lambda t: (0, 0, t))
    return pl.pallas_call(
        _idx_body,
        grid=(NPIX_PAD // BLK_P,),
        in_specs=[spec, spec, spec],
        out_specs=pl.BlockSpec((NZ, BLK_P), lambda t: (0, t)),
        out_shape=jax.ShapeDtypeStruct((NZ, NPIX_PAD), jnp.int32),
    )(u, v, z)


_SC_MESH = plsc.VectorSubcoreMesh(core_axis_name="c", subcore_axis_name="s",
                                  num_cores=2, num_subcores=16)


SLOTS_PER_W = NZ * NPIX_PAD // NWORK
NT = SLOTS_PER_W // CHUNK
NB = 2
UCAP = 144
NCH_P = NPIX_PAD // CHUNK


@functools.partial(
    pl.kernel,
    out_type=(jax.ShapeDtypeStruct((NZ, NCH_P, CHUNK, CPAD), jnp.float32),
              jax.ShapeDtypeStruct((NZ, NCH_P, 1, CHUNK), jnp.int32)),
    mesh=_SC_MESH,
    compiler_params=pltpu.CompilerParams(needs_layout_passes=False),
    scratch_types=[
        pltpu.VMEM((SLOTS_PER_W,), jnp.int32),
        pltpu.VMEM((UCAP,), jnp.int32),
        pltpu.VMEM((NB, 1, CHUNK), jnp.int32),
        pltpu.VMEM((NB, CHUNK, CPAD), jnp.float32),
        pltpu.SemaphoreType.DMA,
        pltpu.SemaphoreType.DMA,
    ],
)
def _sc_gather(table_hbm, idx_hbm, grows_hbm, ord_hbm, idx_v, uq_v, ordb_v,
               rows_v, gsem, osem):
    wid = lax.axis_index("s") * 2 + lax.axis_index("c")
    z0 = wid // 8
    pixbase = (wid % 8) * SLOTS_PER_W
    pb0 = (wid % 8) * NT
    i16 = lax.iota(jnp.int32, 16)
    for q in range(UCAP // 16):
        uq_v[pl.ds(16 * q, 16)] = jnp.zeros((16,), jnp.int32)
    pltpu.sync_copy(idx_hbm.at[z0, pl.ds(pixbase, SLOTS_PER_W)], idx_v)

    ocps = [None] * NB
    o2ps = [None] * NB
    for tt in range(NT // NB):
        for b in range(NB):
            t = tt * NB + b
            if ocps[b] is not None:
                ocps[b].wait()
                o2ps[b].wait()
            nr = jnp.int32(0)
            for k in range(CHUNK // 16):
                off = t * CHUNK + k * 16
                vec = idx_v[pl.ds(off, 16)]
                pidx = jnp.maximum(jnp.full((16,), off - 1, jnp.int32) + i16, 0)
                prev = plsc.load_gather(idx_v, [pidx])
                newrun = vec != prev
                if k == 0:
                    newrun = newrun | (i16 == 0)
                nri = newrun.astype(jnp.int32)
                ordl = nr + plsc.cumsum(nri) - 1
                plsc.store_scatter(uq_v, [ordl], vec, mask=newrun)
                ordb_v[b, 0, pl.ds(k * 16, 16)] = ordl
                nr = nr + jnp.sum(nri)
            for k2 in range(CHUNK // 16):
                @pl.when(k2 * 16 < nr)
                def _():
                    pltpu.async_copy(
                        table_hbm.at[uq_v.at[pl.ds(k2 * 16, 16)]],
                        rows_v.at[b, pl.ds(k2 * 16, 16)], gsem).wait()
            ocps[b] = pltpu.async_copy(rows_v.at[b],
                                       grows_hbm.at[z0, pb0 + t], osem)
            o2ps[b] = pltpu.async_copy(ordb_v.at[b],
                                       ord_hbm.at[z0, pb0 + t], osem)
    for b in range(NB):
        ocps[b].wait()
        o2ps[b].wait()


def _combine_body(grows_ref, ord_ref, w_ref, eye_ref, bias_ref, out_ref):
    w2 = w_ref[...]
    u_iota = lax.broadcasted_iota(jnp.int32, (CHUNK, CHUNK), 0)
    acc = jnp.zeros((CHUNK, CPAD), jnp.float32)
    for z in range(NZ):
        ordz = ord_ref[z, 0]
        oh_t = (u_iota == ordz).astype(jnp.float32)
        m_t = oh_t * w2[z:z + 1, :]
        acc = acc + lax.dot_general(m_t, grows_ref[z, 0],
                                    (((0,), (0,)), ((), ())),
                                    preferred_element_type=jnp.float32)
    out_t = lax.dot_general(eye_ref[...], acc, (((1,), (1,)), ((), ())),
                            preferred_element_type=jnp.float32)
    out_ref[...] = jnp.maximum(out_t + bias_ref[...], 0.0)


def _combine(grows, ords, w_pad, eye, bias_col):
    return pl.pallas_call(
        _combine_body,
        grid=(313,),
        in_specs=[
            pl.BlockSpec((NZ, 1, CHUNK, CPAD), lambda pb: (0, pb, 0, 0)),
            pl.BlockSpec((NZ, 1, 1, CHUNK), lambda pb: (0, pb, 0, 0)),
            pl.BlockSpec((NZ, CHUNK), lambda pb: (0, pb)),
            pl.BlockSpec((COUT, CPAD), lambda pb: (0, 0)),
            pl.BlockSpec((COUT, 1), lambda pb: (0, 0)),
        ],
        out_specs=pl.BlockSpec((COUT, CHUNK), lambda pb: (0, pb)),
        out_shape=jax.ShapeDtypeStruct((COUT, NPIX), jnp.float32),
    )(grows, ords, w_pad, eye, bias_col)


def kernel(mlvl_feats, points, ori_points, img, lidar2camera, lidar2image,
           cam_intrinsic, cam_2_lidar, img_aug_matrix, lidar_aug_matrix,
           img_metas, conv_w, conv_b, bn_gamma, bn_beta, bn_mean, bn_var):
    iam = img_aug_matrix[0]
    iam_r = iam.at[..., -1].set(0.0)
    proj = jnp.matmul(jnp.matmul(iam_r, lidar2image[0]),
                      lidar_aug_matrix[0])[:, :3, :]
    iam_t = iam[..., -1]

    pixv = jnp.arange(NPIX_PAD, dtype=jnp.int32)
    gx = (pixv // NY).astype(jnp.float32) * 0.5 - 50.0
    gy = (pixv % NY).astype(jnp.float32) * 0.5 - 50.0
    gz = jnp.arange(NZ, dtype=jnp.float32) * 1.5 - 4.0
    xf = jnp.broadcast_to(gx[None, :], (NZ, NPIX_PAD)).reshape(-1)
    yf = jnp.broadcast_to(gy[None, :], (NZ, NPIX_PAD)).reshape(-1)
    zf = jnp.broadcast_to(gz[:, None], (NZ, NPIX_PAD)).reshape(-1)
    pts = jnp.stack([xf, yf, zf, jnp.ones_like(xf)], axis=0)
    p2i = jnp.einsum('nij,jk->nik', proj, pts)
    zc = p2i[:, 2]
    uc = p2i[:, 0] / zc + iam_t[:, 0:1]
    vc = p2i[:, 1] / zc + iam_t[:, 1:2]
    u3 = uc.reshape(NCAM, NZ, NPIX_PAD)
    v3 = vc.reshape(NCAM, NZ, NPIX_PAD)
    z3 = zc.reshape(NCAM, NZ, NPIX_PAD)

    s = bn_gamma / jnp.sqrt(bn_var + 1e-5)
    bias_eff = (conv_b - bn_mean) * s + bn_beta
    cw_s = conv_w.T * s[None, :]
    cw_s = jnp.pad(cw_s, ((0, 0), (0, CPAD - COUT)))

    feats_r = mlvl_feats[0].reshape(NCAM, CIN, HW)
    w_pad = jnp.pad(points[0].reshape(NZ, NPIX),
                    ((0, 0), (0, NPIX_PAD - NPIX)))

    table = _pf_table(feats_r, cw_s)
    idx = _idx_map(u3, v3, z3)
    grows, ords = _sc_gather(table, idx)
    eye = jnp.eye(COUT, CPAD, dtype=jnp.float32)
    y = _combine(grows, ords, w_pad, eye, bias_eff[:, None])
    return y.reshape(1, COUT, NX, NY)
